# baseline (device time: 7008 ns/iter reference)
import jax
import jax.numpy as jnp
from jax import lax
from jax.experimental import pallas as pl
from jax.experimental.pallas import tpu as pltpu

N_GLOBAL = 1024
EPS = 1e-5


def kernel(x, gamma):
    m, n = x.shape
    assert m % 128 == 0
    mq = m // 128
    x3 = x.reshape(mq, 128, n)
    gamma3 = gamma.reshape(1, 1, n)

    def body(x_ref, g_ref, out_ref, comm_ref, send_sem, recv_sem):
        my_x = lax.axis_index("x")
        my_y = lax.axis_index("y")
        peer = (my_x, 1 - my_y)

        barrier_sem = pltpu.get_barrier_semaphore()
        pl.semaphore_signal(
            barrier_sem, inc=1, device_id=peer,
            device_id_type=pl.DeviceIdType.MESH,
        )
        pl.semaphore_wait(barrier_sem, 1)

        xf = x_ref[:, :, :].astype(jnp.float32)
        comm_ref[0, :, :] = jnp.sum(xf * xf, axis=-1)

        rdma = pltpu.make_async_remote_copy(
            src_ref=comm_ref.at[0],
            dst_ref=comm_ref.at[1],
            send_sem=send_sem,
            recv_sem=recv_sem,
            device_id=peer,
            device_id_type=pl.DeviceIdType.MESH,
        )
        rdma.start()
        out_ref[:, :, :] = (xf * g_ref[:, :, :].astype(jnp.float32)).astype(
            out_ref.dtype
        )
        rdma.wait()

        total = comm_ref[0, :, :] + comm_ref[1, :, :]
        inv = lax.rsqrt(total / N_GLOBAL + EPS)
        out_ref[:, :, :] = out_ref[:, :, :] * inv[:, :, None].astype(out_ref.dtype)

    out3 = pl.pallas_call(
        body,
        out_shape=jax.ShapeDtypeStruct((mq, 128, n), jnp.bfloat16),
        in_specs=[
            pl.BlockSpec(memory_space=pltpu.VMEM),
            pl.BlockSpec(memory_space=pltpu.VMEM),
        ],
        out_specs=pl.BlockSpec(memory_space=pltpu.VMEM),
        scratch_shapes=[
            pltpu.VMEM((2, mq, 128), jnp.float32),
            pltpu.SemaphoreType.DMA,
            pltpu.SemaphoreType.DMA,
        ],
        compiler_params=pltpu.CompilerParams(collective_id=0),
    )(x3, gamma3)
    return out3.reshape(m, n)


# device time: 6965 ns/iter; 1.0062x vs baseline; 1.0062x over previous
import jax
import jax.numpy as jnp
from jax import lax
from jax.experimental import pallas as pl
from jax.experimental.pallas import tpu as pltpu

N_GLOBAL = 1024
EPS = 1e-5


def kernel(x, gamma):
    m, n = x.shape
    assert m % 128 == 0
    mq = m // 128
    x3 = x.reshape(mq, 128, n)
    gamma3 = gamma.reshape(1, 1, n)

    def body(x_ref, g_ref, out_ref, comm_ref, send_sem, recv_sem):
        my_x = lax.axis_index("x")
        my_y = lax.axis_index("y")
        peer = (my_x, 1 - my_y)

        barrier_sem = pltpu.get_barrier_semaphore()
        pl.semaphore_signal(
            barrier_sem, inc=1, device_id=peer,
            device_id_type=pl.DeviceIdType.MESH,
        )

        xf = x_ref[:, :, :].astype(jnp.float32)
        comm_ref[0, :, :] = jnp.sum(xf * xf, axis=-1)

        pl.semaphore_wait(barrier_sem, 1)
        rdma = pltpu.make_async_remote_copy(
            src_ref=comm_ref.at[0],
            dst_ref=comm_ref.at[1],
            send_sem=send_sem,
            recv_sem=recv_sem,
            device_id=peer,
            device_id_type=pl.DeviceIdType.MESH,
        )
        rdma.start()
        out_ref[:, :, :] = (xf * g_ref[:, :, :].astype(jnp.float32)).astype(
            out_ref.dtype
        )
        rdma.wait_recv()

        total = comm_ref[0, :, :] + comm_ref[1, :, :]
        inv = lax.rsqrt(total / N_GLOBAL + EPS)
        out_ref[:, :, :] = out_ref[:, :, :] * inv[:, :, None].astype(out_ref.dtype)
        rdma.wait_send()

    out3 = pl.pallas_call(
        body,
        out_shape=jax.ShapeDtypeStruct((mq, 128, n), jnp.bfloat16),
        in_specs=[
            pl.BlockSpec(memory_space=pltpu.VMEM),
            pl.BlockSpec(memory_space=pltpu.VMEM),
        ],
        out_specs=pl.BlockSpec(memory_space=pltpu.VMEM),
        scratch_shapes=[
            pltpu.VMEM((2, mq, 128), jnp.float32),
            pltpu.SemaphoreType.DMA,
            pltpu.SemaphoreType.DMA,
        ],
        compiler_params=pltpu.CompilerParams(collective_id=0),
    )(x3, gamma3)
    return out3.reshape(m, n)
